# baseline (speedup 1.0000x reference)
import jax
import jax.numpy as jnp
from jax import lax
from jax.experimental import pallas as pl
from jax.experimental.pallas import tpu as pltpu
from jax.experimental.pallas import tpu_sc as plsc

_B = 16384
_T = 1000

_info = plsc.get_sparse_core_info()
_NC, _NS, _L = _info.num_cores, _info.num_subcores, _info.num_lanes
_NCU = 1
_NW = _NCU * _NS
_BPW = _B // _NW


def _gather_body(t_hbm, beta_hbm, alpha_hbm, out_b_hbm, out_a_hbm,
                 idx_v, beta_v, alpha_v, outb_v, outa_v, sem):
    wid = lax.axis_index("s") * _NCU + lax.axis_index("c")
    base = wid * _BPW
    half = _BPW // 2
    c_idx0 = pltpu.async_copy(
        t_hbm.at[pl.ds(base, half)], idx_v.at[pl.ds(0, half)], sem)
    c_idx1 = pltpu.async_copy(
        t_hbm.at[pl.ds(base + half, half)], idx_v.at[pl.ds(half, half)], sem)
    c_beta = pltpu.async_copy(beta_hbm, beta_v, sem)
    c_alpha = pltpu.async_copy(alpha_hbm, alpha_v, sem)
    c_idx0.wait()
    c_beta.wait()
    c_alpha.wait()

    @plsc.parallel_loop(0, half, _L, unroll=8)
    def _gather_lo(i):
        sl = pl.ds(i, _L)
        idx = idx_v[sl]
        outb_v[sl] = plsc.load_gather(beta_v, [idx])
        outa_v[sl] = plsc.load_gather(alpha_v, [idx])

    o_b0 = pltpu.async_copy(
        outb_v.at[pl.ds(0, half)], out_b_hbm.at[pl.ds(base, half)], sem)
    o_a0 = pltpu.async_copy(
        outa_v.at[pl.ds(0, half)], out_a_hbm.at[pl.ds(base, half)], sem)
    c_idx1.wait()

    @plsc.parallel_loop(half, _BPW, _L, unroll=8)
    def _gather_hi(i):
        sl = pl.ds(i, _L)
        idx = idx_v[sl]
        outb_v[sl] = plsc.load_gather(beta_v, [idx])
        outa_v[sl] = plsc.load_gather(alpha_v, [idx])

    o_b1 = pltpu.async_copy(
        outb_v.at[pl.ds(half, half)], out_b_hbm.at[pl.ds(base + half, half)],
        sem)
    o_a1 = pltpu.async_copy(
        outa_v.at[pl.ds(half, half)], out_a_hbm.at[pl.ds(base + half, half)],
        sem)
    o_b0.wait()
    o_a0.wait()
    o_b1.wait()
    o_a1.wait()


@jax.jit
def _run(t, beta, alpha_cum):
    mesh = plsc.VectorSubcoreMesh(core_axis_name="c", subcore_axis_name="s", num_cores=_NCU)
    fn = pl.kernel(
        _gather_body,
        mesh=mesh,
        out_type=(
            jax.ShapeDtypeStruct((_B,), jnp.float32),
            jax.ShapeDtypeStruct((_B,), jnp.float32),
        ),
        scratch_types=[
            pltpu.VMEM((_BPW,), jnp.int32),
            pltpu.VMEM((_T,), jnp.float32),
            pltpu.VMEM((_T,), jnp.float32),
            pltpu.VMEM((_BPW,), jnp.float32),
            pltpu.VMEM((_BPW,), jnp.float32),
            pltpu.SemaphoreType.DMA,
        ],
        compiler_params=pltpu.CompilerParams(
            needs_layout_passes=False, use_tc_tiling_on_sc=False),
    )
    return fn(t, beta, alpha_cum)


def kernel(t, beta, alpha_cum):
    return _run(t.astype(jnp.int32), beta, alpha_cum)


# speedup vs baseline: 1.0164x; 1.0164x over previous
import jax
import jax.numpy as jnp
from jax import lax
from jax.experimental import pallas as pl
from jax.experimental.pallas import tpu as pltpu
from jax.experimental.pallas import tpu_sc as plsc

_B = 16384
_T = 1000

_info = plsc.get_sparse_core_info()
_NC, _NS, _L = _info.num_cores, _info.num_subcores, _info.num_lanes
_NCU = 1
_NW = _NCU * _NS
_BPW = _B // _NW


def _gather_body(t_hbm, beta_hbm, alpha_hbm, out_b_hbm, out_a_hbm,
                 idx_v, beta_v, alpha_v, outb_v, outa_v, sem):
    wid = lax.axis_index("s") * _NCU + lax.axis_index("c")
    base = wid * _BPW
    c_idx = pltpu.async_copy(t_hbm.at[pl.ds(base, _BPW)], idx_v, sem)
    c_beta = pltpu.async_copy(beta_hbm, beta_v, sem)
    c_alpha = pltpu.async_copy(alpha_hbm, alpha_v, sem)
    c_idx.wait()
    c_beta.wait()
    c_alpha.wait()

    @plsc.parallel_loop(0, _BPW, _L, unroll=8)
    def _gather_iter(i):
        sl = pl.ds(i, _L)
        idx = idx_v[sl]
        outb_v[sl] = plsc.load_gather(beta_v, [idx])
        outa_v[sl] = plsc.load_gather(alpha_v, [idx])

    o_b = pltpu.async_copy(outb_v, out_b_hbm.at[pl.ds(base, _BPW)], sem)
    o_a = pltpu.async_copy(outa_v, out_a_hbm.at[pl.ds(base, _BPW)], sem)
    o_b.wait()
    o_a.wait()


@jax.jit
def _run(t, beta, alpha_cum):
    mesh = plsc.VectorSubcoreMesh(core_axis_name="c", subcore_axis_name="s", num_cores=_NCU)
    fn = pl.kernel(
        _gather_body,
        mesh=mesh,
        out_type=(
            jax.ShapeDtypeStruct((_B,), jnp.float32),
            jax.ShapeDtypeStruct((_B,), jnp.float32),
        ),
        scratch_types=[
            pltpu.VMEM((_BPW,), jnp.int32),
            pltpu.VMEM((_T,), jnp.float32),
            pltpu.VMEM((_T,), jnp.float32),
            pltpu.VMEM((_BPW,), jnp.float32),
            pltpu.VMEM((_BPW,), jnp.float32),
            pltpu.SemaphoreType.DMA,
        ],
        compiler_params=pltpu.CompilerParams(
            needs_layout_passes=False, use_tc_tiling_on_sc=False),
    )
    return fn(t, beta, alpha_cum)


def kernel(t, beta, alpha_cum):
    return _run(t.astype(jnp.int32), beta, alpha_cum)
